# baseline (speedup 1.0000x reference)
import functools

import jax
import jax.numpy as jnp
import numpy as np
from jax import lax
from jax.experimental import pallas as pl
from jax.experimental.pallas import tpu as pltpu
from jax.experimental.pallas import tpu_sc as plsc

N = 10000
H = 256
HH = 128
E = 160000
NP = 10240
NPC = 10112
EPAD = 196608
NSUB = 16
NCORE = 2
TE = EPAD // NSUB
NCH = TE // 128
RPTA = NP // NSUB
RPT = NPC // NSUB

_mesh = plsc.VectorSubcoreMesh(
    core_axis_name="c", subcore_axis_name="s",
    num_cores=NCORE, num_subcores=NSUB)


def _hist_combine(hist_v, cb_v, out_v, sp_hist, s):
    pltpu.sync_copy(hist_v, sp_hist.at[s])
    plsc.subcore_barrier()
    rbase = s * RPTA
    for tt in range(NSUB):
        pltpu.sync_copy(sp_hist.at[tt, pl.ds(rbase, RPTA)], cb_v.at[tt])

    def cbody(j, carry):
        sl = pl.ds(j * 16, 16)
        acc = cb_v[0, sl]
        for tt in range(1, NSUB):
            acc = acc + cb_v[tt, sl]
        out_v[sl] = acc
        return carry
    lax.fori_loop(0, RPTA // 16, cbody, 0)
    return rbase


def _deg1_body(dst_hbm, zeros_hbm, deg_hbm,
               dst_v, hist_v, cb_v, out_v, sp_hist):
    c = lax.axis_index("c")
    s = lax.axis_index("s")
    pltpu.sync_copy(dst_hbm.at[pl.ds(s * TE, TE)], dst_v)
    pltpu.sync_copy(zeros_hbm, hist_v)
    ones = jnp.ones((16,), jnp.float32)

    def body(i, carry):
        idx = dst_v[pl.ds(i * 16, 16)]
        plsc.addupdate_scatter(hist_v, [idx], ones)
        return carry
    lax.fori_loop(0, TE // 16, body, 0)
    rbase = _hist_combine(hist_v, cb_v, out_v, sp_hist, s)

    @pl.when(c == 0)
    def _():
        pltpu.sync_copy(out_v, deg_hbm.at[pl.ds(rbase, RPTA)])


_deg1_kernel = pl.kernel(
    _deg1_body,
    out_type=[jax.ShapeDtypeStruct((NP,), jnp.float32)],
    mesh=_mesh,
    compiler_params=pltpu.CompilerParams(needs_layout_passes=False),
    scratch_types=[pltpu.VMEM((TE,), jnp.int32),
                   pltpu.VMEM((NP,), jnp.float32),
                   pltpu.VMEM((NSUB, RPTA), jnp.float32),
                   pltpu.VMEM((RPTA,), jnp.float32),
                   pltpu.VMEM_SHARED((NSUB, NP), jnp.float32)],
)


def _deg2_body(dst_hbm, src_hbm, perm_hbm, zeros_hbm,
               degp_hbm, psrc_hbm,
               dst_v, src_v, perm_v, psrc_v, hist_v, cb_v, out_v, sp_hist):
    c = lax.axis_index("c")
    s = lax.axis_index("s")
    base = s * TE
    pltpu.sync_copy(perm_hbm, perm_v)
    pltpu.sync_copy(zeros_hbm, hist_v)
    ones = jnp.ones((16,), jnp.float32)

    @pl.when(c == 0)
    def _():
        pltpu.sync_copy(dst_hbm.at[pl.ds(base, TE)], dst_v)

        def body(i, carry):
            sl = pl.ds(i * 16, 16)
            pdst = plsc.load_gather(perm_v, [dst_v[sl]])
            plsc.addupdate_scatter(hist_v, [pdst], ones)
            return carry
        lax.fori_loop(0, TE // 16, body, 0)

    @pl.when(c == 1)
    def _():
        pltpu.sync_copy(src_hbm.at[pl.ds(base, TE)], src_v)

        def body(i, carry):
            sl = pl.ds(i * 16, 16)
            psrc_v[sl] = plsc.load_gather(perm_v, [src_v[sl]])
            return carry
        lax.fori_loop(0, TE // 16, body, 0)
        pltpu.sync_copy(psrc_v, psrc_hbm.at[pl.ds(base, TE)])

    rbase = _hist_combine(hist_v, cb_v, out_v, sp_hist, s)

    @pl.when(c == 0)
    def _():
        pltpu.sync_copy(out_v, degp_hbm.at[pl.ds(rbase, RPTA)])


_deg2_kernel = pl.kernel(
    _deg2_body,
    out_type=[jax.ShapeDtypeStruct((NP,), jnp.float32),
              jax.ShapeDtypeStruct((EPAD,), jnp.int32)],
    mesh=_mesh,
    compiler_params=pltpu.CompilerParams(needs_layout_passes=False),
    scratch_types=[pltpu.VMEM((TE,), jnp.int32),
                   pltpu.VMEM((TE,), jnp.int32),
                   pltpu.VMEM((NP,), jnp.int32),
                   pltpu.VMEM((TE,), jnp.int32),
                   pltpu.VMEM((NP,), jnp.float32),
                   pltpu.VMEM((NSUB, RPTA), jnp.float32),
                   pltpu.VMEM((RPTA,), jnp.float32),
                   pltpu.VMEM_SHARED((NSUB, NP), jnp.float32)],
)


CPR = 48
RNDS = NCH // CPR


def _agg_body(hcat, gix_hbm, dst3, zer_hbm, acc4,
              ixq, dxq, bufs, sems, acc_sh):
    c = lax.axis_index("c")
    s = lax.axis_index("s")
    rbase = s * RPT

    def phase_body(p, carry):
        slot = p * 2 + c
        pltpu.sync_copy(zer_hbm, acc_sh.at[pl.ds(rbase, RPT)])
        plsc.subcore_barrier()

        def round_body(r, carry2):
            pltpu.sync_copy(gix_hbm.at[slot, s, pl.ds(r * CPR, CPR)], ixq)
            pltpu.sync_copy(dst3.at[s, pl.ds(r * CPR, CPR)], dxq)

            def prime(k, carry3):
                pltpu.async_copy(hcat.at[ixq.at[k]], bufs.at[k], sems.at[k])
                return carry3
            lax.fori_loop(0, 2, prime, 0)

            def body(k, carry3):
                par = lax.rem(k, 2)
                pltpu.make_async_copy(hcat.at[ixq.at[k]], bufs.at[par],
                                      sems.at[par]).wait()
                pltpu.sync_copy(bufs.at[par], acc_sh.at[dxq.at[k]], add=True)

                @pl.when(k + 2 < CPR)
                def _():
                    pltpu.async_copy(hcat.at[ixq.at[k + 2]], bufs.at[par],
                                     sems.at[par])
                return carry3
            lax.fori_loop(0, CPR, body, 0)
            return carry2
        lax.fori_loop(0, RNDS, round_body, 0)
        plsc.subcore_barrier()
        pltpu.sync_copy(acc_sh.at[pl.ds(rbase, RPT)],
                        acc4.at[slot, pl.ds(rbase, RPT)])
        return carry

    lax.fori_loop(0, 2, phase_body, 0)


_agg_kernel = pl.kernel(
    _agg_body,
    out_type=[jax.ShapeDtypeStruct((4, NPC, HH), jnp.float32)],
    mesh=_mesh,
    compiler_params=pltpu.CompilerParams(needs_layout_passes=False),
    scratch_types=[pltpu.VMEM((CPR, 128), jnp.int32),
                   pltpu.VMEM((CPR, 128), jnp.int32),
                   pltpu.VMEM((2, 128, HH), jnp.float32),
                   pltpu.SemaphoreType.DMA((2,)),
                   pltpu.VMEM_SHARED((NPC, HH), jnp.float32)],
)


_BBLK = 2000


def _enc_body(x_ref, w_ref, deg_ref, degp_ref, out_ref):
    h = jnp.dot(x_ref[...], w_ref[...], preferred_element_type=jnp.float32)
    dinv = lax.rsqrt(deg_ref[...])
    dinvp = lax.rsqrt(degp_ref[...])
    hp = h * dinv
    hq = h * dinvp
    out_ref[...] = jnp.stack(
        [hp[:, :HH], hp[:, HH:], hq[:, :HH], hq[:, HH:]])


def _enc_call(x, W, deg2, degp2):
    grid = (N // _BBLK,)
    return pl.pallas_call(
        _enc_body,
        grid=grid,
        in_specs=[
            pl.BlockSpec((_BBLK, H), lambda i: (i, 0)),
            pl.BlockSpec((H, H), lambda i: (0, 0)),
            pl.BlockSpec((_BBLK, 1), lambda i: (i, 0)),
            pl.BlockSpec((_BBLK, 1), lambda i: (i, 0)),
        ],
        out_specs=pl.BlockSpec((4, _BBLK, HH), lambda i: (0, i, 0)),
        out_shape=jax.ShapeDtypeStruct((4, N, HH), jnp.float32),
    )(x, W, deg2, degp2)


_DBLK = 2000
_DNB = N // _DBLK


def _loss_body(a0, a1, a2_, a3, deg_ref, b_ref, a_ref, w_ref, out_ref,
               pos_s, neg_s, S_s, v_s, l1_s, l2_s):
    p = pl.program_id(0)
    i = pl.program_id(1)
    rows = pl.ds(i * _DBLK, _DBLK)

    @pl.when(p == 0)
    def _():
        dinv = lax.rsqrt(deg_ref[...])
        b = b_ref[...]
        a = a_ref[...]
        accp = jnp.concatenate([a0[0], a1[0]], axis=1)
        outp = accp * dinv + b
        pos = jnp.where(outp > 0, outp, a * outp)
        pos_s[rows, :] = pos
        accn = jnp.concatenate([a2_[0], a3[0]], axis=1)
        outn = accn * dinv + b
        neg_s[rows, :] = jnp.where(outn > 0, outn, a * outn)

        @pl.when(i == 0)
        def _():
            S_s[...] = jnp.zeros_like(S_s)

        S_s[...] += jnp.sum(pos, axis=0, keepdims=True)

    @pl.when(p == 1)
    def _():
        @pl.when(i == 0)
        def _():
            summary = jax.nn.sigmoid(S_s[...] / N)
            v_s[...] = jax.lax.dot_general(
                summary, w_ref[...], (((1,), (1,)), ((), ())),
                preferred_element_type=jnp.float32)
            l1_s[0, 0] = 0.0
            l2_s[0, 0] = 0.0

        v = v_s[...]
        lp = jax.lax.dot_general(pos_s[rows, :], v, (((1,), (1,)), ((), ())),
                                 preferred_element_type=jnp.float32)
        ln = jax.lax.dot_general(neg_s[rows, :], v, (((1,), (1,)), ((), ())),
                                 preferred_element_type=jnp.float32)
        l1_s[0, 0] += jnp.sum(jnp.logaddexp(0.0, -lp))
        l2_s[0, 0] += jnp.sum(jnp.logaddexp(0.0, ln))

        @pl.when(i == _DNB - 1)
        def _():
            out_ref[...] = jnp.full(
                (1, 1), (l1_s[0, 0] + l2_s[0, 0]) / N, jnp.float32)


def _loss_call(acc4, deg2, b2, a2, disc_W):
    grid = (2, _DNB)

    def _slot(k):
        return pl.BlockSpec((1, _DBLK, HH),
                            lambda p, i, k=k: (k, i * (1 - p) + (_DNB - 1) * p, 0))

    return pl.pallas_call(
        _loss_body,
        grid=grid,
        in_specs=[_slot(k) for k in range(4)] + [
            pl.BlockSpec((_DBLK, 1), lambda p, i: (i * (1 - p) + (_DNB - 1) * p, 0)),
            pl.BlockSpec((1, H), lambda p, i: (0, 0)),
            pl.BlockSpec((1, H), lambda p, i: (0, 0)),
            pl.BlockSpec((H, H), lambda p, i: (0, 0)),
        ],
        out_specs=pl.BlockSpec((1, 1), lambda p, i: (0, 0)),
        out_shape=jax.ShapeDtypeStruct((1, 1), jnp.float32),
        scratch_shapes=[
            pltpu.VMEM((N, H), jnp.float32),
            pltpu.VMEM((N, H), jnp.float32),
            pltpu.VMEM((1, H), jnp.float32),
            pltpu.VMEM((1, H), jnp.float32),
            pltpu.SMEM((1, 1), jnp.float32),
            pltpu.SMEM((1, 1), jnp.float32),
        ],
    )(*([acc4] * 4), deg2, b2, a2, disc_W)


def kernel(x, edge_index, W_gcn, b_gcn, prelu_a, disc_W):
    perm = jax.random.permutation(jax.random.key(1), N).astype(jnp.int32)
    src = edge_index[0].astype(jnp.int32)
    dst = edge_index[1].astype(jnp.int32)
    npad = EPAD - E - N
    iota = np.arange(N, dtype=np.int32)
    pad_src = np.arange(npad, dtype=np.int32) % N
    pad_dst = (N + np.arange(npad, dtype=np.int32) % (NPC - N)).astype(np.int32)
    src_all = jnp.concatenate([src, jnp.asarray(iota), jnp.asarray(pad_src)])
    dst_all = jnp.concatenate([dst, jnp.asarray(iota), jnp.asarray(pad_dst)])
    perm_pad = jnp.concatenate(
        [perm, jnp.asarray(N + np.arange(NP - N, dtype=np.int32))])
    zeros1d = jnp.zeros((NP,), jnp.float32)

    deg, = _deg1_kernel(dst_all, zeros1d)
    degp, psrc_all = _deg2_kernel(dst_all, src_all, perm_pad, zeros1d)
    deg2 = deg.reshape(NP, 1)
    degp2 = degp.reshape(NP, 1)

    hcat = _enc_call(x, W_gcn, deg2, degp2).reshape(4 * N, HH)

    gix = jnp.stack([src_all, src_all + N,
                     psrc_all + 2 * N, psrc_all + 3 * N
                     ]).reshape(4, NSUB, NCH, 128)
    dst3 = dst_all.reshape(NSUB, NCH, 128)
    zer = jnp.zeros((RPT, HH), jnp.float32)
    acc4, = _agg_kernel(hcat, gix, dst3, zer)

    loss = _loss_call(acc4, deg2, b_gcn.reshape(1, H), prelu_a.reshape(1, H),
                      disc_W)
    return loss[0, 0]


# speedup vs baseline: 1.0302x; 1.0302x over previous
import functools

import jax
import jax.numpy as jnp
import numpy as np
from jax import lax
from jax.experimental import pallas as pl
from jax.experimental.pallas import tpu as pltpu
from jax.experimental.pallas import tpu_sc as plsc

N = 10000
H = 256
HH = 128
E = 160000
NP = 10240
NPC = 10112
EPAD = 196608
NSUB = 16
NCORE = 2
TE = EPAD // NSUB
NCH = TE // 128
RPTA = NP // NSUB
RPT = NPC // NSUB

_mesh = plsc.VectorSubcoreMesh(
    core_axis_name="c", subcore_axis_name="s",
    num_cores=NCORE, num_subcores=NSUB)


def _hist_combine(hist_v, cb_v, out_v, sp_hist, s):
    pltpu.sync_copy(hist_v, sp_hist.at[s])
    plsc.subcore_barrier()
    rbase = s * RPTA
    for tt in range(NSUB):
        pltpu.sync_copy(sp_hist.at[tt, pl.ds(rbase, RPTA)], cb_v.at[tt])

    def cbody(j, carry):
        sl = pl.ds(j * 16, 16)
        acc = cb_v[0, sl]
        for tt in range(1, NSUB):
            acc = acc + cb_v[tt, sl]
        out_v[sl] = acc
        return carry
    lax.fori_loop(0, RPTA // 16, cbody, 0)
    return rbase


def _deg_body(dst_hbm, src_hbm, perm_hbm, zeros_hbm,
              deg_hbm, degp_hbm, psrc_hbm,
              dst_v, src_v, perm_v, psrc_v, hist_v, cb_v, out_v, sp_hist):
    c = lax.axis_index("c")
    s = lax.axis_index("s")
    base = s * TE
    pltpu.sync_copy(dst_hbm.at[pl.ds(base, TE)], dst_v)
    pltpu.sync_copy(zeros_hbm, hist_v)
    ones = jnp.ones((16,), jnp.float32)

    @pl.when(c == 0)
    def _():
        def body(i, carry):
            idx = dst_v[pl.ds(i * 16, 16)]
            plsc.addupdate_scatter(hist_v, [idx], ones)
            return carry
        lax.fori_loop(0, TE // 16, body, 0)

    @pl.when(c == 1)
    def _():
        pltpu.sync_copy(src_hbm.at[pl.ds(base, TE)], src_v)
        pltpu.sync_copy(perm_hbm, perm_v)

        def body(i, carry):
            sl = pl.ds(i * 16, 16)
            pdst = plsc.load_gather(perm_v, [dst_v[sl]])
            plsc.addupdate_scatter(hist_v, [pdst], ones)
            psrc_v[sl] = plsc.load_gather(perm_v, [src_v[sl]])
            return carry
        lax.fori_loop(0, TE // 16, body, 0)
        pltpu.sync_copy(psrc_v, psrc_hbm.at[pl.ds(base, TE)])

    rbase = _hist_combine(hist_v, cb_v, out_v, sp_hist, s)

    @pl.when(c == 0)
    def _():
        pltpu.sync_copy(out_v, deg_hbm.at[pl.ds(rbase, RPTA)])

    @pl.when(c == 1)
    def _():
        pltpu.sync_copy(out_v, degp_hbm.at[pl.ds(rbase, RPTA)])


_deg_kernel = pl.kernel(
    _deg_body,
    out_type=[jax.ShapeDtypeStruct((NP,), jnp.float32),
              jax.ShapeDtypeStruct((NP,), jnp.float32),
              jax.ShapeDtypeStruct((EPAD,), jnp.int32)],
    mesh=_mesh,
    compiler_params=pltpu.CompilerParams(needs_layout_passes=False),
    scratch_types=[pltpu.VMEM((TE,), jnp.int32),
                   pltpu.VMEM((TE,), jnp.int32),
                   pltpu.VMEM((NP,), jnp.int32),
                   pltpu.VMEM((TE,), jnp.int32),
                   pltpu.VMEM((NP,), jnp.float32),
                   pltpu.VMEM((NSUB, RPTA), jnp.float32),
                   pltpu.VMEM((RPTA,), jnp.float32),
                   pltpu.VMEM_SHARED((NSUB, NP), jnp.float32)],
)


CPR = 48
RNDS = NCH // CPR


def _agg_body(hcat, gix_hbm, dst3, zer_hbm, acc4,
              ixq, dxq, bufs, sems, acc_sh):
    c = lax.axis_index("c")
    s = lax.axis_index("s")
    rbase = s * RPT

    def phase_body(p, carry):
        slot = p * 2 + c
        pltpu.sync_copy(zer_hbm, acc_sh.at[pl.ds(rbase, RPT)])
        plsc.subcore_barrier()

        def round_body(r, carry2):
            pltpu.sync_copy(gix_hbm.at[slot, s, pl.ds(r * CPR, CPR)], ixq)
            pltpu.sync_copy(dst3.at[s, pl.ds(r * CPR, CPR)], dxq)

            def prime(k, carry3):
                pltpu.async_copy(hcat.at[ixq.at[k]], bufs.at[k], sems.at[k])
                return carry3
            lax.fori_loop(0, 2, prime, 0)

            def body(k, carry3):
                par = lax.rem(k, 2)
                pltpu.make_async_copy(hcat.at[ixq.at[k]], bufs.at[par],
                                      sems.at[par]).wait()
                pltpu.sync_copy(bufs.at[par], acc_sh.at[dxq.at[k]], add=True)

                @pl.when(k + 2 < CPR)
                def _():
                    pltpu.async_copy(hcat.at[ixq.at[k + 2]], bufs.at[par],
                                     sems.at[par])
                return carry3
            lax.fori_loop(0, CPR, body, 0)
            return carry2
        lax.fori_loop(0, RNDS, round_body, 0)
        plsc.subcore_barrier()
        pltpu.sync_copy(acc_sh.at[pl.ds(rbase, RPT)],
                        acc4.at[slot, pl.ds(rbase, RPT)])
        return carry

    lax.fori_loop(0, 2, phase_body, 0)


_agg_kernel = pl.kernel(
    _agg_body,
    out_type=[jax.ShapeDtypeStruct((4, NPC, HH), jnp.float32)],
    mesh=_mesh,
    compiler_params=pltpu.CompilerParams(needs_layout_passes=False),
    scratch_types=[pltpu.VMEM((CPR, 128), jnp.int32),
                   pltpu.VMEM((CPR, 128), jnp.int32),
                   pltpu.VMEM((2, 128, HH), jnp.float32),
                   pltpu.SemaphoreType.DMA((2,)),
                   pltpu.VMEM_SHARED((NPC, HH), jnp.float32)],
)


_BBLK = 2000


def _enc_body(x_ref, w_ref, deg_ref, degp_ref, out_ref):
    h = jnp.dot(x_ref[...], w_ref[...], preferred_element_type=jnp.float32)
    dinv = lax.rsqrt(deg_ref[...])
    dinvp = lax.rsqrt(degp_ref[...])
    hp = h * dinv
    hq = h * dinvp
    out_ref[...] = jnp.stack(
        [hp[:, :HH], hp[:, HH:], hq[:, :HH], hq[:, HH:]])


def _enc_call(x, W, deg2, degp2):
    grid = (N // _BBLK,)
    return pl.pallas_call(
        _enc_body,
        grid=grid,
        in_specs=[
            pl.BlockSpec((_BBLK, H), lambda i: (i, 0)),
            pl.BlockSpec((H, H), lambda i: (0, 0)),
            pl.BlockSpec((_BBLK, 1), lambda i: (i, 0)),
            pl.BlockSpec((_BBLK, 1), lambda i: (i, 0)),
        ],
        out_specs=pl.BlockSpec((4, _BBLK, HH), lambda i: (0, i, 0)),
        out_shape=jax.ShapeDtypeStruct((4, N, HH), jnp.float32),
    )(x, W, deg2, degp2)


_DBLK = 2000
_DNB = N // _DBLK


def _loss_body(a0, a1, a2_, a3, deg_ref, b_ref, a_ref, w_ref, out_ref,
               pos_s, neg_s, S_s, v_s, l1_s, l2_s):
    p = pl.program_id(0)
    i = pl.program_id(1)
    rows = pl.ds(i * _DBLK, _DBLK)

    @pl.when(p == 0)
    def _():
        dinv = lax.rsqrt(deg_ref[...])
        b = b_ref[...]
        a = a_ref[...]
        accp = jnp.concatenate([a0[0], a1[0]], axis=1)
        outp = accp * dinv + b
        pos = jnp.where(outp > 0, outp, a * outp)
        pos_s[rows, :] = pos
        accn = jnp.concatenate([a2_[0], a3[0]], axis=1)
        outn = accn * dinv + b
        neg_s[rows, :] = jnp.where(outn > 0, outn, a * outn)

        @pl.when(i == 0)
        def _():
            S_s[...] = jnp.zeros_like(S_s)

        S_s[...] += jnp.sum(pos, axis=0, keepdims=True)

    @pl.when(p == 1)
    def _():
        @pl.when(i == 0)
        def _():
            summary = jax.nn.sigmoid(S_s[...] / N)
            v_s[...] = jax.lax.dot_general(
                summary, w_ref[...], (((1,), (1,)), ((), ())),
                preferred_element_type=jnp.float32)
            l1_s[0, 0] = 0.0
            l2_s[0, 0] = 0.0

        v = v_s[...]
        lp = jax.lax.dot_general(pos_s[rows, :], v, (((1,), (1,)), ((), ())),
                                 preferred_element_type=jnp.float32)
        ln = jax.lax.dot_general(neg_s[rows, :], v, (((1,), (1,)), ((), ())),
                                 preferred_element_type=jnp.float32)
        l1_s[0, 0] += jnp.sum(jnp.logaddexp(0.0, -lp))
        l2_s[0, 0] += jnp.sum(jnp.logaddexp(0.0, ln))

        @pl.when(i == _DNB - 1)
        def _():
            out_ref[...] = jnp.full(
                (1, 1), (l1_s[0, 0] + l2_s[0, 0]) / N, jnp.float32)


def _loss_call(acc4, deg2, b2, a2, disc_W):
    grid = (2, _DNB)

    def _slot(k):
        return pl.BlockSpec((1, _DBLK, HH),
                            lambda p, i, k=k: (k, i * (1 - p) + (_DNB - 1) * p, 0))

    return pl.pallas_call(
        _loss_body,
        grid=grid,
        in_specs=[_slot(k) for k in range(4)] + [
            pl.BlockSpec((_DBLK, 1), lambda p, i: (i * (1 - p) + (_DNB - 1) * p, 0)),
            pl.BlockSpec((1, H), lambda p, i: (0, 0)),
            pl.BlockSpec((1, H), lambda p, i: (0, 0)),
            pl.BlockSpec((H, H), lambda p, i: (0, 0)),
        ],
        out_specs=pl.BlockSpec((1, 1), lambda p, i: (0, 0)),
        out_shape=jax.ShapeDtypeStruct((1, 1), jnp.float32),
        scratch_shapes=[
            pltpu.VMEM((N, H), jnp.float32),
            pltpu.VMEM((N, H), jnp.float32),
            pltpu.VMEM((1, H), jnp.float32),
            pltpu.VMEM((1, H), jnp.float32),
            pltpu.SMEM((1, 1), jnp.float32),
            pltpu.SMEM((1, 1), jnp.float32),
        ],
    )(*([acc4] * 4), deg2, b2, a2, disc_W)


def kernel(x, edge_index, W_gcn, b_gcn, prelu_a, disc_W):
    perm = jax.random.permutation(jax.random.key(1), N).astype(jnp.int32)
    src = edge_index[0].astype(jnp.int32)
    dst = edge_index[1].astype(jnp.int32)
    npad = EPAD - E - N
    iota = np.arange(N, dtype=np.int32)
    pad_src = np.arange(npad, dtype=np.int32) % N
    pad_dst = (N + np.arange(npad, dtype=np.int32) % (NPC - N)).astype(np.int32)
    src_all = jnp.concatenate([src, jnp.asarray(iota), jnp.asarray(pad_src)])
    dst_all = jnp.concatenate([dst, jnp.asarray(iota), jnp.asarray(pad_dst)])
    perm_pad = jnp.concatenate(
        [perm, jnp.asarray(N + np.arange(NP - N, dtype=np.int32))])
    zeros1d = jnp.zeros((NP,), jnp.float32)

    deg, degp, psrc_all = _deg_kernel(dst_all, src_all, perm_pad, zeros1d)
    deg2 = deg.reshape(NP, 1)
    degp2 = degp.reshape(NP, 1)

    hcat = _enc_call(x, W_gcn, deg2, degp2).reshape(4 * N, HH)

    gix = jnp.stack([src_all, src_all + N,
                     psrc_all + 2 * N, psrc_all + 3 * N
                     ]).reshape(4, NSUB, NCH, 128)
    dst3 = dst_all.reshape(NSUB, NCH, 128)
    zer = jnp.zeros((RPT, HH), jnp.float32)
    acc4, = _agg_kernel(hcat, gix, dst3, zer)

    loss = _loss_call(acc4, deg2, b_gcn.reshape(1, H), prelu_a.reshape(1, H),
                      disc_W)
    return loss[0, 0]


# speedup vs baseline: 1.1091x; 1.0766x over previous
import functools

import jax
import jax.numpy as jnp
import numpy as np
from jax import lax
from jax.experimental import pallas as pl
from jax.experimental.pallas import tpu as pltpu
from jax.experimental.pallas import tpu_sc as plsc

N = 10000
H = 256
HH = 128
E = 160000
NP = 10240
NPC = 10112
EPAD = 180224
NSUB = 16
NCORE = 2
TE = EPAD // NSUB
NCH = TE // 128
RPTA = NP // NSUB
RPT = NPC // NSUB

_mesh = plsc.VectorSubcoreMesh(
    core_axis_name="c", subcore_axis_name="s",
    num_cores=NCORE, num_subcores=NSUB)


def _hist_combine(hist_v, cb_v, out_v, sp_hist, s):
    pltpu.sync_copy(hist_v, sp_hist.at[s])
    plsc.subcore_barrier()
    rbase = s * RPTA
    for tt in range(NSUB):
        pltpu.sync_copy(sp_hist.at[tt, pl.ds(rbase, RPTA)], cb_v.at[tt])

    def cbody(j, carry):
        sl = pl.ds(j * 16, 16)
        acc = cb_v[0, sl]
        for tt in range(1, NSUB):
            acc = acc + cb_v[tt, sl]
        out_v[sl] = acc
        return carry
    lax.fori_loop(0, RPTA // 16, cbody, 0)
    return rbase


def _deg_body(dst_hbm, src_hbm, perm_hbm, zeros_hbm,
              deg_hbm, degp_hbm, psrc_hbm,
              dst_v, src_v, perm_v, psrc_v, hist_v, cb_v, out_v, sp_hist):
    c = lax.axis_index("c")
    s = lax.axis_index("s")
    base = s * TE
    pltpu.sync_copy(dst_hbm.at[pl.ds(base, TE)], dst_v)
    pltpu.sync_copy(zeros_hbm, hist_v)
    ones = jnp.ones((16,), jnp.float32)

    @pl.when(c == 0)
    def _():
        def body(i, carry):
            idx = dst_v[pl.ds(i * 16, 16)]
            plsc.addupdate_scatter(hist_v, [idx], ones)
            return carry
        lax.fori_loop(0, TE // 16, body, 0)

    @pl.when(c == 1)
    def _():
        pltpu.sync_copy(src_hbm.at[pl.ds(base, TE)], src_v)
        pltpu.sync_copy(perm_hbm, perm_v)

        def body(i, carry):
            sl = pl.ds(i * 16, 16)
            pdst = plsc.load_gather(perm_v, [dst_v[sl]])
            plsc.addupdate_scatter(hist_v, [pdst], ones)
            psrc_v[sl] = plsc.load_gather(perm_v, [src_v[sl]])
            return carry
        lax.fori_loop(0, TE // 16, body, 0)
        pltpu.sync_copy(psrc_v, psrc_hbm.at[pl.ds(base, TE)])

    rbase = _hist_combine(hist_v, cb_v, out_v, sp_hist, s)

    @pl.when(c == 0)
    def _():
        pltpu.sync_copy(out_v, deg_hbm.at[pl.ds(rbase, RPTA)])

    @pl.when(c == 1)
    def _():
        pltpu.sync_copy(out_v, degp_hbm.at[pl.ds(rbase, RPTA)])


_deg_kernel = pl.kernel(
    _deg_body,
    out_type=[jax.ShapeDtypeStruct((NP,), jnp.float32),
              jax.ShapeDtypeStruct((NP,), jnp.float32),
              jax.ShapeDtypeStruct((EPAD,), jnp.int32)],
    mesh=_mesh,
    compiler_params=pltpu.CompilerParams(needs_layout_passes=False),
    scratch_types=[pltpu.VMEM((TE,), jnp.int32),
                   pltpu.VMEM((TE,), jnp.int32),
                   pltpu.VMEM((NP,), jnp.int32),
                   pltpu.VMEM((TE,), jnp.int32),
                   pltpu.VMEM((NP,), jnp.float32),
                   pltpu.VMEM((NSUB, RPTA), jnp.float32),
                   pltpu.VMEM((RPTA,), jnp.float32),
                   pltpu.VMEM_SHARED((NSUB, NP), jnp.float32)],
)


CPR = 8
RNDS = NCH // CPR


def _agg_body(hcat, gix_hbm, dst3, zer_hbm, acc4,
              ixq, dxv, bufs, sems, ssem, acc_sh):
    c = lax.axis_index("c")
    s = lax.axis_index("s")
    rbase = s * RPT
    pltpu.sync_copy(dst3.at[s], dxv)

    def phase_body(p, carry):
        slot = p * 2 + c
        pltpu.sync_copy(zer_hbm, acc_sh.at[pl.ds(rbase, RPT)])
        plsc.subcore_barrier()
        pltpu.sync_copy(gix_hbm.at[slot, s, pl.ds(0, CPR)], ixq.at[0])
        pltpu.async_copy(gix_hbm.at[slot, s, pl.ds(CPR, CPR)], ixq.at[1],
                         ssem)

        def prime(k, carry2):
            pltpu.async_copy(hcat.at[ixq.at[0, k]], bufs.at[k], sems.at[k])
            return carry2
        lax.fori_loop(0, 2, prime, 0)

        def body(k, carry2):
            r = lax.div(k, CPR)
            j = lax.rem(k, CPR)
            rp = lax.rem(r, 2)
            par = lax.rem(k, 2)
            k2 = k + 2
            pltpu.make_async_copy(hcat.at[ixq.at[rp, j]], bufs.at[par],
                                  sems.at[par]).wait()
            pltpu.sync_copy(bufs.at[par], acc_sh.at[dxv.at[k]], add=True)

            @pl.when((j == CPR - 2) & (r + 1 < RNDS))
            def _():
                pltpu.make_async_copy(
                    gix_hbm.at[slot, s, pl.ds((r + 1) * CPR, CPR)],
                    ixq.at[1 - rp], ssem).wait()

            @pl.when((j == CPR - 2) & (r + 2 < RNDS))
            def _():
                pltpu.async_copy(
                    gix_hbm.at[slot, s, pl.ds((r + 2) * CPR, CPR)],
                    ixq.at[rp], ssem)

            @pl.when(k2 < NCH)
            def _():
                r2 = lax.div(k2, CPR)
                j2 = lax.rem(k2, CPR)
                pltpu.async_copy(hcat.at[ixq.at[lax.rem(r2, 2), j2]],
                                 bufs.at[par], sems.at[par])
            return carry2
        lax.fori_loop(0, NCH, body, 0)
        plsc.subcore_barrier()
        pltpu.sync_copy(acc_sh.at[pl.ds(rbase, RPT)],
                        acc4.at[slot, pl.ds(rbase, RPT)])
        return carry

    lax.fori_loop(0, 2, phase_body, 0)


_agg_kernel = pl.kernel(
    _agg_body,
    out_type=[jax.ShapeDtypeStruct((4, NPC, HH), jnp.float32)],
    mesh=_mesh,
    compiler_params=pltpu.CompilerParams(needs_layout_passes=False),
    scratch_types=[pltpu.VMEM((2, CPR, 128), jnp.int32),
                   pltpu.VMEM((NCH, 128), jnp.int32),
                   pltpu.VMEM((2, 128, HH), jnp.float32),
                   pltpu.SemaphoreType.DMA((2,)),
                   pltpu.SemaphoreType.DMA,
                   pltpu.VMEM_SHARED((NPC, HH), jnp.float32)],
)


_BBLK = 2000


def _enc_body(x_ref, w_ref, deg_ref, degp_ref, out_ref):
    h = jnp.dot(x_ref[...], w_ref[...], preferred_element_type=jnp.float32)
    dinv = lax.rsqrt(deg_ref[...])
    dinvp = lax.rsqrt(degp_ref[...])
    hp = h * dinv
    hq = h * dinvp
    out_ref[...] = jnp.stack(
        [hp[:, :HH], hp[:, HH:], hq[:, :HH], hq[:, HH:]])


def _enc_call(x, W, deg2, degp2):
    grid = (N // _BBLK,)
    return pl.pallas_call(
        _enc_body,
        grid=grid,
        in_specs=[
            pl.BlockSpec((_BBLK, H), lambda i: (i, 0)),
            pl.BlockSpec((H, H), lambda i: (0, 0)),
            pl.BlockSpec((_BBLK, 1), lambda i: (i, 0)),
            pl.BlockSpec((_BBLK, 1), lambda i: (i, 0)),
        ],
        out_specs=pl.BlockSpec((4, _BBLK, HH), lambda i: (0, i, 0)),
        out_shape=jax.ShapeDtypeStruct((4, N, HH), jnp.float32),
    )(x, W, deg2, degp2)


_DBLK = 2000
_DNB = N // _DBLK


def _loss_body(a0, a1, a2_, a3, deg_ref, b_ref, a_ref, w_ref, out_ref,
               pos_s, neg_s, S_s, v_s, l1_s, l2_s):
    p = pl.program_id(0)
    i = pl.program_id(1)
    rows = pl.ds(i * _DBLK, _DBLK)

    @pl.when(p == 0)
    def _():
        dinv = lax.rsqrt(deg_ref[...])
        b = b_ref[...]
        a = a_ref[...]
        accp = jnp.concatenate([a0[0], a1[0]], axis=1)
        outp = accp * dinv + b
        pos = jnp.where(outp > 0, outp, a * outp)
        pos_s[rows, :] = pos
        accn = jnp.concatenate([a2_[0], a3[0]], axis=1)
        outn = accn * dinv + b
        neg_s[rows, :] = jnp.where(outn > 0, outn, a * outn)

        @pl.when(i == 0)
        def _():
            S_s[...] = jnp.zeros_like(S_s)

        S_s[...] += jnp.sum(pos, axis=0, keepdims=True)

    @pl.when(p == 1)
    def _():
        @pl.when(i == 0)
        def _():
            summary = jax.nn.sigmoid(S_s[...] / N)
            v_s[...] = jax.lax.dot_general(
                summary, w_ref[...], (((1,), (1,)), ((), ())),
                preferred_element_type=jnp.float32)
            l1_s[0, 0] = 0.0
            l2_s[0, 0] = 0.0

        v = v_s[...]
        lp = jax.lax.dot_general(pos_s[rows, :], v, (((1,), (1,)), ((), ())),
                                 preferred_element_type=jnp.float32)
        ln = jax.lax.dot_general(neg_s[rows, :], v, (((1,), (1,)), ((), ())),
                                 preferred_element_type=jnp.float32)
        l1_s[0, 0] += jnp.sum(jnp.logaddexp(0.0, -lp))
        l2_s[0, 0] += jnp.sum(jnp.logaddexp(0.0, ln))

        @pl.when(i == _DNB - 1)
        def _():
            out_ref[...] = jnp.full(
                (1, 1), (l1_s[0, 0] + l2_s[0, 0]) / N, jnp.float32)


def _loss_call(acc4, deg2, b2, a2, disc_W):
    grid = (2, _DNB)

    def _slot(k):
        return pl.BlockSpec((1, _DBLK, HH),
                            lambda p, i, k=k: (k, i * (1 - p) + (_DNB - 1) * p, 0))

    return pl.pallas_call(
        _loss_body,
        grid=grid,
        in_specs=[_slot(k) for k in range(4)] + [
            pl.BlockSpec((_DBLK, 1), lambda p, i: (i * (1 - p) + (_DNB - 1) * p, 0)),
            pl.BlockSpec((1, H), lambda p, i: (0, 0)),
            pl.BlockSpec((1, H), lambda p, i: (0, 0)),
            pl.BlockSpec((H, H), lambda p, i: (0, 0)),
        ],
        out_specs=pl.BlockSpec((1, 1), lambda p, i: (0, 0)),
        out_shape=jax.ShapeDtypeStruct((1, 1), jnp.float32),
        scratch_shapes=[
            pltpu.VMEM((N, H), jnp.float32),
            pltpu.VMEM((N, H), jnp.float32),
            pltpu.VMEM((1, H), jnp.float32),
            pltpu.VMEM((1, H), jnp.float32),
            pltpu.SMEM((1, 1), jnp.float32),
            pltpu.SMEM((1, 1), jnp.float32),
        ],
    )(*([acc4] * 4), deg2, b2, a2, disc_W)


def kernel(x, edge_index, W_gcn, b_gcn, prelu_a, disc_W):
    perm = jax.random.permutation(jax.random.key(1), N).astype(jnp.int32)
    src = edge_index[0].astype(jnp.int32)
    dst = edge_index[1].astype(jnp.int32)
    npad = EPAD - E - N
    iota = np.arange(N, dtype=np.int32)
    pad_src = np.arange(npad, dtype=np.int32) % N
    pad_dst = (N + np.arange(npad, dtype=np.int32) % (NPC - N)).astype(np.int32)
    src_all = jnp.concatenate([src, jnp.asarray(iota), jnp.asarray(pad_src)])
    dst_all = jnp.concatenate([dst, jnp.asarray(iota), jnp.asarray(pad_dst)])
    perm_pad = jnp.concatenate(
        [perm, jnp.asarray(N + np.arange(NP - N, dtype=np.int32))])
    zeros1d = jnp.zeros((NP,), jnp.float32)

    deg, degp, psrc_all = _deg_kernel(dst_all, src_all, perm_pad, zeros1d)
    deg2 = deg.reshape(NP, 1)
    degp2 = degp.reshape(NP, 1)

    hcat = _enc_call(x, W_gcn, deg2, degp2).reshape(4 * N, HH)

    gix = jnp.stack([src_all, src_all + N,
                     psrc_all + 2 * N, psrc_all + 3 * N
                     ]).reshape(4, NSUB, NCH, 128)
    dst3 = dst_all.reshape(NSUB, NCH, 128)
    zer = jnp.zeros((RPT, HH), jnp.float32)
    acc4, = _agg_kernel(hcat, gix, dst3, zer)

    loss = _loss_call(acc4, deg2, b_gcn.reshape(1, H), prelu_a.reshape(1, H),
                      disc_W)
    return loss[0, 0]


# speedup vs baseline: 1.1093x; 1.0002x over previous
import jax
import jax.numpy as jnp
import numpy as np
from jax import lax
from jax.experimental import pallas as pl
from jax.experimental.pallas import tpu as pltpu
from jax.experimental.pallas import tpu_sc as plsc

N = 10000
H = 256
HH = 128
E = 160000
NP = 10240
NPC = 10112
EPAD = 180224
NSUB = 16
NCORE = 2
TE = EPAD // NSUB
NCH = TE // 128
RPTA = NP // NSUB
RPT = NPC // NSUB

_mesh = plsc.VectorSubcoreMesh(
    core_axis_name="c", subcore_axis_name="s",
    num_cores=NCORE, num_subcores=NSUB)


def _hist_combine(hist_v, cb_v, out_v, sp_hist, s):
    pltpu.sync_copy(hist_v, sp_hist.at[s])
    plsc.subcore_barrier()
    rbase = s * RPTA
    for tt in range(NSUB):
        pltpu.sync_copy(sp_hist.at[tt, pl.ds(rbase, RPTA)], cb_v.at[tt])

    def cbody(j, carry):
        sl = pl.ds(j * 16, 16)
        acc = cb_v[0, sl]
        for tt in range(1, NSUB):
            acc = acc + cb_v[tt, sl]
        out_v[sl] = acc
        return carry
    lax.fori_loop(0, RPTA // 16, cbody, 0)
    return rbase


def _deg_body(dst_hbm, src_hbm, perm_hbm, zeros_hbm,
              deg_hbm, degp_hbm, psrc_hbm,
              dst_v, src_v, perm_v, psrc_v, hist_v, cb_v, out_v, sp_hist):
    c = lax.axis_index("c")
    s = lax.axis_index("s")
    base = s * TE
    pltpu.sync_copy(dst_hbm.at[pl.ds(base, TE)], dst_v)
    pltpu.sync_copy(zeros_hbm, hist_v)
    ones = jnp.ones((16,), jnp.float32)

    @pl.when(c == 0)
    def _():
        def body(i, carry):
            idx = dst_v[pl.ds(i * 16, 16)]
            plsc.addupdate_scatter(hist_v, [idx], ones)
            return carry
        lax.fori_loop(0, TE // 16, body, 0)

    @pl.when(c == 1)
    def _():
        pltpu.sync_copy(src_hbm.at[pl.ds(base, TE)], src_v)
        pltpu.sync_copy(perm_hbm, perm_v)

        def body(i, carry):
            sl = pl.ds(i * 16, 16)
            pdst = plsc.load_gather(perm_v, [dst_v[sl]])
            plsc.addupdate_scatter(hist_v, [pdst], ones)
            psrc_v[sl] = plsc.load_gather(perm_v, [src_v[sl]])
            return carry
        lax.fori_loop(0, TE // 16, body, 0)
        pltpu.sync_copy(psrc_v, psrc_hbm.at[pl.ds(base, TE)])

    rbase = _hist_combine(hist_v, cb_v, out_v, sp_hist, s)

    @pl.when(c == 0)
    def _():
        pltpu.sync_copy(out_v, deg_hbm.at[pl.ds(rbase, RPTA)])

    @pl.when(c == 1)
    def _():
        pltpu.sync_copy(out_v, degp_hbm.at[pl.ds(rbase, RPTA)])


_deg_kernel = pl.kernel(
    _deg_body,
    out_type=[jax.ShapeDtypeStruct((NP,), jnp.float32),
              jax.ShapeDtypeStruct((NP,), jnp.float32),
              jax.ShapeDtypeStruct((EPAD,), jnp.int32)],
    mesh=_mesh,
    compiler_params=pltpu.CompilerParams(needs_layout_passes=False),
    scratch_types=[pltpu.VMEM((TE,), jnp.int32),
                   pltpu.VMEM((TE,), jnp.int32),
                   pltpu.VMEM((NP,), jnp.int32),
                   pltpu.VMEM((TE,), jnp.int32),
                   pltpu.VMEM((NP,), jnp.float32),
                   pltpu.VMEM((NSUB, RPTA), jnp.float32),
                   pltpu.VMEM((RPTA,), jnp.float32),
                   pltpu.VMEM_SHARED((NSUB, NP), jnp.float32)],
)


CPR = 8
RNDS = NCH // CPR


def _agg_body(hcat, gix_hbm, dst3, zer_hbm, acc4,
              ixq, dxv, bufs, sems, ssem, acc_sh):
    c = lax.axis_index("c")
    s = lax.axis_index("s")
    rbase = s * RPT
    pltpu.sync_copy(dst3.at[s], dxv)

    def phase_body(p, carry):
        slot = p * 2 + c
        pltpu.sync_copy(zer_hbm, acc_sh.at[pl.ds(rbase, RPT)])
        plsc.subcore_barrier()
        pltpu.sync_copy(gix_hbm.at[slot, s, pl.ds(0, CPR)], ixq.at[0])
        pltpu.async_copy(gix_hbm.at[slot, s, pl.ds(CPR, CPR)], ixq.at[1],
                         ssem)

        def prime(k, carry2):
            pltpu.async_copy(hcat.at[ixq.at[0, k]], bufs.at[k], sems.at[k])
            return carry2
        lax.fori_loop(0, 2, prime, 0)

        def body(k, carry2):
            r = lax.div(k, CPR)
            j = lax.rem(k, CPR)
            rp = lax.rem(r, 2)
            par = lax.rem(k, 2)
            k2 = k + 2
            pltpu.make_async_copy(hcat.at[ixq.at[rp, j]], bufs.at[par],
                                  sems.at[par]).wait()
            pltpu.sync_copy(bufs.at[par], acc_sh.at[dxv.at[k]], add=True)

            @pl.when((j == CPR - 2) & (r + 1 < RNDS))
            def _():
                pltpu.make_async_copy(
                    gix_hbm.at[slot, s, pl.ds((r + 1) * CPR, CPR)],
                    ixq.at[1 - rp], ssem).wait()

            @pl.when((j == CPR - 2) & (r + 2 < RNDS))
            def _():
                pltpu.async_copy(
                    gix_hbm.at[slot, s, pl.ds((r + 2) * CPR, CPR)],
                    ixq.at[rp], ssem)

            @pl.when(k2 < NCH)
            def _():
                r2 = lax.div(k2, CPR)
                j2 = lax.rem(k2, CPR)
                pltpu.async_copy(hcat.at[ixq.at[lax.rem(r2, 2), j2]],
                                 bufs.at[par], sems.at[par])
            return carry2
        lax.fori_loop(0, NCH, body, 0)
        plsc.subcore_barrier()
        pltpu.sync_copy(acc_sh.at[pl.ds(rbase, RPT)],
                        acc4.at[slot, pl.ds(rbase, RPT)])
        return carry

    lax.fori_loop(0, 2, phase_body, 0)


_agg_kernel = pl.kernel(
    _agg_body,
    out_type=[jax.ShapeDtypeStruct((4, NPC, HH), jnp.float32)],
    mesh=_mesh,
    compiler_params=pltpu.CompilerParams(needs_layout_passes=False),
    scratch_types=[pltpu.VMEM((2, CPR, 128), jnp.int32),
                   pltpu.VMEM((NCH, 128), jnp.int32),
                   pltpu.VMEM((2, 128, HH), jnp.float32),
                   pltpu.SemaphoreType.DMA((2,)),
                   pltpu.SemaphoreType.DMA,
                   pltpu.VMEM_SHARED((NPC, HH), jnp.float32)],
)


_BBLK = 2000


def _enc_body(x_ref, w_ref, deg_ref, degp_ref, out_ref):
    h = jnp.dot(x_ref[...], w_ref[...], preferred_element_type=jnp.float32)
    dinv = lax.rsqrt(deg_ref[...])
    dinvp = lax.rsqrt(degp_ref[...])
    hp = h * dinv
    hq = h * dinvp
    out_ref[...] = jnp.stack(
        [hp[:, :HH], hp[:, HH:], hq[:, :HH], hq[:, HH:]])


def _enc_call(x, W, deg2, degp2):
    grid = (N // _BBLK,)
    return pl.pallas_call(
        _enc_body,
        grid=grid,
        in_specs=[
            pl.BlockSpec((_BBLK, H), lambda i: (i, 0)),
            pl.BlockSpec((H, H), lambda i: (0, 0)),
            pl.BlockSpec((_BBLK, 1), lambda i: (i, 0)),
            pl.BlockSpec((_BBLK, 1), lambda i: (i, 0)),
        ],
        out_specs=pl.BlockSpec((4, _BBLK, HH), lambda i: (0, i, 0)),
        out_shape=jax.ShapeDtypeStruct((4, N, HH), jnp.float32),
    )(x, W, deg2, degp2)


_DBLK = 2000
_DNB = N // _DBLK


def _loss_body(a0, a1, a2_, a3, deg_ref, b_ref, a_ref, w_ref, out_ref,
               pos_s, neg_s, S_s, v_s, l1_s, l2_s):
    p = pl.program_id(0)
    i = pl.program_id(1)
    rows = pl.ds(i * _DBLK, _DBLK)

    @pl.when(p == 0)
    def _():
        dinv = lax.rsqrt(deg_ref[...])
        b = b_ref[...]
        a = a_ref[...]
        accp = jnp.concatenate([a0[0], a1[0]], axis=1)
        outp = accp * dinv + b
        pos = jnp.where(outp > 0, outp, a * outp)
        pos_s[rows, :] = pos
        accn = jnp.concatenate([a2_[0], a3[0]], axis=1)
        outn = accn * dinv + b
        neg_s[rows, :] = jnp.where(outn > 0, outn, a * outn)

        @pl.when(i == 0)
        def _():
            S_s[...] = jnp.zeros_like(S_s)

        S_s[...] += jnp.sum(pos, axis=0, keepdims=True)

    @pl.when(p == 1)
    def _():
        @pl.when(i == 0)
        def _():
            summary = jax.nn.sigmoid(S_s[...] / N)
            v_s[...] = jax.lax.dot_general(
                summary, w_ref[...], (((1,), (1,)), ((), ())),
                preferred_element_type=jnp.float32)
            l1_s[0, 0] = 0.0
            l2_s[0, 0] = 0.0

        v = v_s[...]
        lp = jax.lax.dot_general(pos_s[rows, :], v, (((1,), (1,)), ((), ())),
                                 preferred_element_type=jnp.float32)
        ln = jax.lax.dot_general(neg_s[rows, :], v, (((1,), (1,)), ((), ())),
                                 preferred_element_type=jnp.float32)
        l1_s[0, 0] += jnp.sum(jnp.logaddexp(0.0, -lp))
        l2_s[0, 0] += jnp.sum(jnp.logaddexp(0.0, ln))

        @pl.when(i == _DNB - 1)
        def _():
            out_ref[...] = jnp.full(
                (1, 1), (l1_s[0, 0] + l2_s[0, 0]) / N, jnp.float32)


def _loss_call(acc4, deg2, b2, a2, disc_W):
    grid = (2, _DNB)

    def _slot(k):
        return pl.BlockSpec((1, _DBLK, HH),
                            lambda p, i, k=k: (k, i * (1 - p) + (_DNB - 1) * p, 0))

    return pl.pallas_call(
        _loss_body,
        grid=grid,
        in_specs=[_slot(k) for k in range(4)] + [
            pl.BlockSpec((_DBLK, 1), lambda p, i: (i * (1 - p) + (_DNB - 1) * p, 0)),
            pl.BlockSpec((1, H), lambda p, i: (0, 0)),
            pl.BlockSpec((1, H), lambda p, i: (0, 0)),
            pl.BlockSpec((H, H), lambda p, i: (0, 0)),
        ],
        out_specs=pl.BlockSpec((1, 1), lambda p, i: (0, 0)),
        out_shape=jax.ShapeDtypeStruct((1, 1), jnp.float32),
        scratch_shapes=[
            pltpu.VMEM((N, H), jnp.float32),
            pltpu.VMEM((N, H), jnp.float32),
            pltpu.VMEM((1, H), jnp.float32),
            pltpu.VMEM((1, H), jnp.float32),
            pltpu.SMEM((1, 1), jnp.float32),
            pltpu.SMEM((1, 1), jnp.float32),
        ],
    )(*([acc4] * 4), deg2, b2, a2, disc_W)


def kernel(x, edge_index, W_gcn, b_gcn, prelu_a, disc_W):
    perm = jax.random.permutation(jax.random.key(1), N).astype(jnp.int32)
    src = edge_index[0].astype(jnp.int32)
    dst = edge_index[1].astype(jnp.int32)
    npad = EPAD - E - N
    iota = np.arange(N, dtype=np.int32)
    pad_src = np.arange(npad, dtype=np.int32) % N
    pad_dst = (N + np.arange(npad, dtype=np.int32) % (NPC - N)).astype(np.int32)
    src_all = jnp.concatenate([src, jnp.asarray(iota), jnp.asarray(pad_src)])
    dst_all = jnp.concatenate([dst, jnp.asarray(iota), jnp.asarray(pad_dst)])
    perm_pad = jnp.concatenate(
        [perm, jnp.asarray(N + np.arange(NP - N, dtype=np.int32))])
    zeros1d = jnp.zeros((NP,), jnp.float32)

    deg, degp, psrc_all = _deg_kernel(dst_all, src_all, perm_pad, zeros1d)
    deg2 = deg.reshape(NP, 1)
    degp2 = degp.reshape(NP, 1)

    hcat = _enc_call(x, W_gcn, deg2, degp2).reshape(4 * N, HH)

    gix = jnp.stack([src_all, src_all + N,
                     psrc_all + 2 * N, psrc_all + 3 * N
                     ]).reshape(4, NSUB, NCH, 128)
    dst3 = dst_all.reshape(NSUB, NCH, 128)
    zer = jnp.zeros((RPT, HH), jnp.float32)
    acc4, = _agg_kernel(hcat, gix, dst3, zer)

    loss = _loss_call(acc4, deg2, b_gcn.reshape(1, H), prelu_a.reshape(1, H),
                      disc_W)
    return loss[0, 0]
